# baseline (speedup 1.0000x reference)
import functools

import jax
import jax.numpy as jnp
from jax import lax
from jax.experimental import pallas as pl
from jax.experimental.pallas import tpu as pltpu
from jax.experimental.pallas import tpu_sc as plsc

N = 320000
D = 128
B = 1024
NC = 2
NS = 16
NW = NC * NS
RPW = N // NW
GCH = 200
SCH = 100
SPG = GCH // SCH
GPW = RPW // GCH
IDR = RPW // SCH
BPS = B // NS


def _sc_segment_sum(rows_hbm_arr, ids3):
  mesh = plsc.VectorSubcoreMesh(core_axis_name="c", subcore_axis_name="s")

  @functools.partial(
      pl.kernel,
      mesh=mesh,
      out_type=jax.ShapeDtypeStruct((NC, B, D), jnp.float32),
      scratch_types=[
          pltpu.VMEM((IDR, SCH), jnp.int32),
          pltpu.VMEM((GCH, D), jnp.float32),
          pltpu.VMEM((GCH, D), jnp.float32),
          pltpu.VMEM((BPS, D), jnp.float32),
          pltpu.VMEM_SHARED((B, D), jnp.float32),
          pltpu.SemaphoreType.DMA,
          pltpu.SemaphoreType.DMA,
          pltpu.SemaphoreType.DMA,
          pltpu.SemaphoreType.DMA,
      ],
  )
  def k(rows_hbm, ids_hbm, out_hbm, ids_v, b0, b1, tmp_v, acc_sh,
        g0, g1, s0, s1):
    cid = lax.axis_index("c")
    sid = lax.axis_index("s")
    wid = cid * NS + sid
    bufs = (b0, b1)
    gsems = (g0, g1)
    ssems = (s0, s1)

    def zrow(r, carry):
      for c in range(D // 16):
        tmp_v[r, pl.ds(c * 16, 16)] = jnp.zeros((16,), jnp.float32)
      return carry
    lax.fori_loop(0, BPS, zrow, 0)
    pltpu.sync_copy(tmp_v, acc_sh.at[pl.ds(sid * BPS, BPS)])

    pltpu.sync_copy(ids_hbm.at[wid], ids_v)
    plsc.subcore_barrier()

    base = wid * RPW

    def g_start(gi, b):
      pltpu.make_async_copy(rows_hbm.at[pl.ds(base + gi * GCH, GCH)],
                            bufs[b], gsems[b]).start()

    def g_wait(b):
      pltpu.make_async_copy(rows_hbm.at[pl.ds(base, GCH)],
                            bufs[b], gsems[b]).wait()

    def scat_start(gi, b):
      descs = []
      for p in range(SPG):
        descs.append(
            pltpu.async_copy(bufs[b].at[pl.ds(p * SCH, SCH)],
                             acc_sh.at[ids_v.at[gi * SPG + p]], ssems[b],
                             add=True))
      return descs

    g_start(0, 0)
    g_start(1, 1)

    def outer(i, carry):
      gi = i * 2
      g_wait(0)
      d0 = scat_start(gi, 0)
      g_wait(1)
      d1 = scat_start(gi + 1, 1)
      for d in d0:
        d.wait()

      @pl.when(gi + 2 < GPW)
      def _():
        g_start(gi + 2, 0)
      for d in d1:
        d.wait()

      @pl.when(gi + 3 < GPW)
      def _():
        g_start(gi + 3, 1)
      return carry

    lax.fori_loop(0, GPW // 2, outer, 0)

    plsc.subcore_barrier()
    pltpu.sync_copy(acc_sh.at[pl.ds(sid * BPS, BPS)], tmp_v)
    pltpu.sync_copy(tmp_v, out_hbm.at[cid, pl.ds(sid * BPS, BPS)])

  return k(rows_hbm_arr, ids3)


def _combine(partials):
  def add_body(a_ref, b_ref, o_ref):
    o_ref[...] = a_ref[...] + b_ref[...]

  return pl.pallas_call(
      add_body,
      out_shape=jax.ShapeDtypeStruct((B, D), jnp.float32),
  )(partials[0], partials[1])


def kernel(outputs, atom_split):
  ids3 = atom_split.astype(jnp.int32).reshape(NW, IDR, SCH)
  partials = _sc_segment_sum(outputs, ids3)
  return _combine(partials)


# speedup vs baseline: 1.1249x; 1.1249x over previous
import functools

import jax
import jax.numpy as jnp
from jax import lax
from jax.experimental import pallas as pl
from jax.experimental.pallas import tpu as pltpu
from jax.experimental.pallas import tpu_sc as plsc

N = 320000
D = 128
B = 1024
NC = 2
NS = 16
NW = NC * NS
RPW = N // NW
CHUNK = 40
CPW = RPW // CHUNK
NBUF = 10
SPB = CPW // NBUF
BPS = B // NS


def _sc_segment_sum(rows_hbm_arr, ids3):
  mesh = plsc.VectorSubcoreMesh(core_axis_name="c", subcore_axis_name="s")

  scratch = [pltpu.VMEM((CPW, CHUNK), jnp.int32)]
  scratch += [pltpu.VMEM((CHUNK, D), jnp.float32) for _ in range(NBUF)]
  scratch += [pltpu.VMEM((BPS, D), jnp.float32)]
  scratch += [pltpu.VMEM_SHARED((B, D), jnp.float32)]
  scratch += [pltpu.SemaphoreType.DMA for _ in range(2 * NBUF)]

  @functools.partial(
      pl.kernel,
      mesh=mesh,
      out_type=jax.ShapeDtypeStruct((NC, B, D), jnp.float32),
      scratch_types=scratch,
  )
  def k(rows_hbm, ids_hbm, out_hbm, ids_v, *rest):
    bufs = rest[:NBUF]
    tmp_v = rest[NBUF]
    acc_sh = rest[NBUF + 1]
    gsems = rest[NBUF + 2:2 * NBUF + 2]
    ssems = rest[2 * NBUF + 2:]
    cid = lax.axis_index("c")
    sid = lax.axis_index("s")
    wid = cid * NS + sid

    def zrow(r, carry):
      for c in range(D // 16):
        tmp_v[r, pl.ds(c * 16, 16)] = jnp.zeros((16,), jnp.float32)
      return carry
    lax.fori_loop(0, BPS, zrow, 0)
    pltpu.sync_copy(tmp_v, acc_sh.at[pl.ds(sid * BPS, BPS)])

    pltpu.sync_copy(ids_hbm.at[wid], ids_v)
    plsc.subcore_barrier()

    base = wid * RPW

    def g_start(ci, b):
      pltpu.make_async_copy(rows_hbm.at[pl.ds(base + ci * CHUNK, CHUNK)],
                            bufs[b], gsems[b]).start()

    def g_wait(b):
      pltpu.make_async_copy(rows_hbm.at[pl.ds(base, CHUNK)],
                            bufs[b], gsems[b]).wait()

    for b in range(NBUF):
      g_start(b, b)

    def outer(i, carry):
      c0 = i * NBUF
      scatters = []
      for b in range(NBUF):
        g_wait(b)
        scatters.append(
            pltpu.async_copy(bufs[b], acc_sh.at[ids_v.at[c0 + b]], ssems[b],
                             add=True))
      for b in range(NBUF):
        scatters[b].wait()
        nxt = c0 + NBUF + b

        @pl.when(nxt < CPW)
        def _():
          g_start(nxt, b)
      return carry

    lax.fori_loop(0, SPB, outer, 0)

    plsc.subcore_barrier()
    pltpu.sync_copy(acc_sh.at[pl.ds(sid * BPS, BPS)], tmp_v)
    pltpu.sync_copy(tmp_v, out_hbm.at[cid, pl.ds(sid * BPS, BPS)])

  return k(rows_hbm_arr, ids3)


def _combine(partials):
  def add_body(a_ref, b_ref, o_ref):
    o_ref[...] = a_ref[...] + b_ref[...]

  return pl.pallas_call(
      add_body,
      out_shape=jax.ShapeDtypeStruct((B, D), jnp.float32),
  )(partials[0], partials[1])


def kernel(outputs, atom_split):
  ids3 = atom_split.astype(jnp.int32).reshape(NW, CPW, CHUNK)
  partials = _sc_segment_sum(outputs, ids3)
  return _combine(partials)
